# Initial kernel scaffold; baseline (speedup 1.0000x reference)
import jax
import jax.numpy as jnp
from jax.experimental import pallas as pl


def kernel(x, emb, w_out, b_out):
    raise NotImplementedError("write your pallas kernel here")


# speedup vs baseline: 1.1553x; 1.1553x over previous
import functools

import jax
import jax.numpy as jnp
from jax import lax
from jax.experimental import pallas as pl
from jax.experimental.pallas import tpu as pltpu
from jax.experimental.pallas import tpu_sc as plsc

_IGNORE = -100
_V = 100000
_D = 768
_S = 2047
_SP = 2048
_VB = 1024
_NV = (_V + _VB - 1) // _VB

_NEG = -1e30


@functools.cache
def _make_sc_gather():
    info = plsc.get_sparse_core_info()
    nw = info.num_cores * info.num_subcores
    b_per_w = _SP // nw
    mesh = plsc.VectorSubcoreMesh(core_axis_name="c", subcore_axis_name="s")

    @functools.partial(
        pl.kernel,
        mesh=mesh,
        out_type=jax.ShapeDtypeStruct((_SP, _D), jnp.float32),
        scratch_types=[
            pltpu.VMEM((b_per_w,), jnp.int32),
            pltpu.VMEM((b_per_w, _D), jnp.float32),
            pltpu.SemaphoreType.DMA,
        ],
    )
    def gather_k(table_hbm, idx_hbm, out_hbm, idx_v, rows_v, sem):
        wid = lax.axis_index("s") * info.num_cores + lax.axis_index("c")
        base = wid * b_per_w
        pltpu.sync_copy(idx_hbm.at[pl.ds(base, b_per_w)], idx_v)
        pltpu.async_copy(table_hbm.at[idx_v], rows_v, sem).wait()
        pltpu.sync_copy(rows_v, out_hbm.at[pl.ds(base, b_per_w)])

    return gather_k


def _ce_body(tgt_ref, h_ref, w_ref, b_ref, out_ref, m_ref, s_ref, t_ref):
    v = pl.program_id(0)

    @pl.when(v == 0)
    def _init():
        m_ref[...] = jnp.full((_SP, 1), _NEG, jnp.float32)
        s_ref[...] = jnp.zeros((_SP, 1), jnp.float32)
        t_ref[...] = jnp.zeros((_SP, 1), jnp.float32)

    logits = jnp.dot(h_ref[...], w_ref[...],
                     preferred_element_type=jnp.float32) + b_ref[...]
    col = v * _VB + lax.broadcasted_iota(jnp.int32, (_SP, _VB), 1)
    logits = jnp.where(col < _V, logits, _NEG)

    tgt = tgt_ref[...]
    t_ref[...] += jnp.sum(jnp.where(col == tgt, logits, 0.0),
                          axis=1, keepdims=True)

    m_prev = m_ref[...]
    m_new = jnp.maximum(m_prev, jnp.max(logits, axis=1, keepdims=True))
    s_ref[...] = (s_ref[...] * jnp.exp(m_prev - m_new)
                  + jnp.sum(jnp.exp(logits - m_new), axis=1, keepdims=True))
    m_ref[...] = m_new

    @pl.when(v == _NV - 1)
    def _fin():
        lse = m_ref[...] + jnp.log(s_ref[...])
        nll = lse - t_ref[...]
        valid = (tgt >= 0).astype(jnp.float32)
        loss_sum = jnp.sum(nll * valid)
        denom = jnp.maximum(jnp.sum(valid), 1.0)
        out_ref[0, 0] = loss_sum / denom


def _ce_loss(tgt2d, h, w_out, b2d, interpret=False):
    out = pl.pallas_call(
        _ce_body,
        grid=(_NV,),
        in_specs=[
            pl.BlockSpec((_SP, 1), lambda v: (0, 0)),
            pl.BlockSpec((_SP, _D), lambda v: (0, 0)),
            pl.BlockSpec((_D, _VB), lambda v: (0, v)),
            pl.BlockSpec((1, _VB), lambda v: (0, v)),
        ],
        out_specs=pl.BlockSpec((1, 1), lambda v: (0, 0),
                               memory_space=pltpu.SMEM),
        out_shape=jax.ShapeDtypeStruct((1, 1), jnp.float32),
        scratch_shapes=[
            pltpu.VMEM((_SP, 1), jnp.float32),
            pltpu.VMEM((_SP, 1), jnp.float32),
            pltpu.VMEM((_SP, 1), jnp.float32),
        ],
        interpret=interpret,
    )(tgt2d, h, w_out, b2d)
    return out[0, 0]


def kernel(x, emb, w_out, b_out):
    inp = x[0, :-1]
    inp = jnp.where(inp == _IGNORE, 0, inp)
    idx = jnp.pad(inp, (0, _SP - _S))
    tgt = jnp.pad(x[0, 1:], (0, _SP - _S), constant_values=-1)
    tgt2d = tgt.reshape(_SP, 1)
    b2d = b_out.reshape(1, _V)

    h = _make_sc_gather()(emb, idx)
    return _ce_loss(tgt2d, h, w_out, b2d)
